# baseline (speedup 1.0000x reference)
import functools

import jax
import jax.numpy as jnp
from jax import lax
from jax.experimental import pallas as pl
from jax.experimental.pallas import tpu as pltpu
from jax.experimental.pallas import tpu_sc as plsc

_N = 10000
_E = 320000
_D = 128
_EPS = 1e-5

_NC = 2
_NS = 16
_NW = _NC * _NS
_NP = 10240
_RPT = _NP // _NS
_EP = 327680
_EPW = _EP // _NW
_CH = 128
_NCHUNK = _EPW // _CH

_MESH = plsc.VectorSubcoreMesh(
    core_axis_name="c", subcore_axis_name="s", num_cores=_NC, num_subcores=_NS
)


def _worker_ids():
    cid = lax.axis_index("c")
    sid = lax.axis_index("s")
    return cid, sid, sid * _NC + cid


@functools.partial(
    pl.kernel,
    out_type=jax.ShapeDtypeStruct((_NC, 2, _NP), jnp.float32),
    mesh=_MESH,
    scratch_types=[
        pltpu.VMEM((_CH,), jnp.int32),
        pltpu.VMEM((_CH,), jnp.int32),
        pltpu.VMEM((_CH,), jnp.float32),
        pltpu.VMEM_SHARED((_NP,), jnp.float32),
        pltpu.VMEM_SHARED((_NP,), jnp.float32),
    ],
)
def _deg_kernel(src_hbm, dst_hbm, ones_hbm, zeros_hbm, out_hbm,
                sidx, didx, ones_v, acc_s, acc_d):
    cid, sid, wid = _worker_ids()
    rows = pl.ds(sid * _RPT, _RPT)
    pltpu.sync_copy(zeros_hbm, acc_s.at[rows])
    pltpu.sync_copy(zeros_hbm, acc_d.at[rows])
    pltpu.sync_copy(ones_hbm, ones_v)
    plsc.subcore_barrier()

    def body(i, carry):
        off = pl.multiple_of(wid * _EPW + i * _CH, 128)
        pltpu.sync_copy(src_hbm.at[pl.ds(off, _CH)], sidx)
        pltpu.sync_copy(dst_hbm.at[pl.ds(off, _CH)], didx)
        pltpu.sync_copy(ones_v, acc_s.at[sidx], add=True)
        pltpu.sync_copy(ones_v, acc_d.at[didx], add=True)
        return carry

    lax.fori_loop(0, _NCHUNK, body, 0)
    plsc.subcore_barrier()
    pltpu.sync_copy(acc_s.at[rows], out_hbm.at[cid, 0, rows])
    pltpu.sync_copy(acc_d.at[rows], out_hbm.at[cid, 1, rows])


def _deg_call(src, dst):
    ones1 = jnp.ones((_CH,), jnp.float32)
    zeros1 = jnp.zeros((_RPT,), jnp.float32)
    degs = _deg_kernel(src, dst, ones1, zeros1)
    return degs[:, :, :, None]


_S0 = 160
_S1 = _NCHUNK * 2 - _S0
_G = 32


@functools.partial(
    pl.kernel,
    out_type=jax.ShapeDtypeStruct((_NC, _NP, _D), jnp.float32),
    mesh=_MESH,
    scratch_types=[
        pltpu.VMEM((_G, _CH), jnp.int32),
        pltpu.VMEM((_G, _CH), jnp.int32),
        pltpu.VMEM((_CH, _D), jnp.float32),
        pltpu.VMEM((_CH, _D), jnp.float32),
        pltpu.SemaphoreType.DMA,
        pltpu.SemaphoreType.DMA,
        pltpu.VMEM_SHARED((_NP, _D), jnp.float32),
    ],
)
def _edge_kernel(x_hbm, src_hbm, dst_hbm, zeros_hbm, out_hbm,
                 sidx, didx, rows0, rows1, sem0, sem1, acc):
    cid, sid, wid = _worker_ids()
    rows = pl.ds(sid * _RPT, _RPT)
    pltpu.sync_copy(zeros_hbm, acc.at[rows])
    plsc.subcore_barrier()

    def run_groups(tile_start, n_groups):
        for g in range(n_groups):
            chunks = pl.ds(tile_start + g * _G, _G)
            pltpu.sync_copy(src_hbm.at[chunks], sidx)
            pltpu.sync_copy(dst_hbm.at[chunks], didx)
            pltpu.async_copy(x_hbm.at[sidx.at[0]], rows0, sem0)

            def body(i, carry):
                i2 = i * 2
                pltpu.make_async_copy(
                    x_hbm.at[sidx.at[i2]], rows0, sem0).wait()
                pltpu.async_copy(x_hbm.at[sidx.at[i2 + 1]], rows1, sem1)
                pltpu.sync_copy(rows0, acc.at[didx.at[i2]], add=True)
                pltpu.make_async_copy(
                    x_hbm.at[sidx.at[i2 + 1]], rows1, sem1).wait()

                @pl.when(i2 + 2 < _G)
                def _():
                    pltpu.async_copy(x_hbm.at[sidx.at[i2 + 2]], rows0, sem0)

                pltpu.sync_copy(rows1, acc.at[didx.at[i2 + 1]], add=True)
                return carry

            lax.fori_loop(0, _G // 2, body, 0)

    @pl.when(cid == 0)
    def _():
        run_groups(sid * _S0, _S0 // _G)

    if _S1 > 0:
        @pl.when(cid == 1)
        def _():
            run_groups(16 * _S0 + sid * _S1, _S1 // _G)

    plsc.subcore_barrier()
    pltpu.sync_copy(acc.at[rows], out_hbm.at[cid, rows])


def _norms_from(degs, which):
    d = degs[0, which, : _N] + degs[1, which, : _N]
    return lax.rsqrt(jnp.maximum(d, 1.0))


def _prep_body(h_ref, degs_ref, x0_ref):
    degs = degs_ref[...]
    x0_ref[0:_N] = h_ref[...] * _norms_from(degs, 0)
    x0_ref[_N:] = jnp.zeros((_NP - _N, _D), jnp.float32)


def _dense0_body(aggp_ref, degs_ref, w_ref, b_ref, g_ref, be_ref,
                 h1_ref, x1_ref):
    degs = degs_ref[...]
    agg = (aggp_ref[0, : _N] + aggp_ref[1, : _N]) * _norms_from(degs, 1)
    t = jnp.dot(agg, w_ref[...], preferred_element_type=jnp.float32) + b_ref[...]
    m = jnp.mean(t, axis=0, keepdims=True)
    v = jnp.mean((t - m) ** 2, axis=0, keepdims=True)
    h1 = jnp.maximum((t - m) * lax.rsqrt(v + _EPS) * g_ref[...] + be_ref[...], 0.0)
    h1_ref[...] = h1
    x1_ref[0:_N] = h1 * _norms_from(degs, 0)
    x1_ref[_N:] = jnp.zeros((_NP - _N, _D), jnp.float32)


def _final_body(aggp_ref, degs_ref, w_ref, b_ref, g_ref, be_ref, h1_ref,
                wp_ref, bp_ref, score_ref):
    degs = degs_ref[...]
    agg = (aggp_ref[0, : _N] + aggp_ref[1, : _N]) * _norms_from(degs, 1)
    t = jnp.dot(agg, w_ref[...], preferred_element_type=jnp.float32) + b_ref[...]
    m = jnp.mean(t, axis=0, keepdims=True)
    v = jnp.mean((t - m) ** 2, axis=0, keepdims=True)
    h2 = jnp.maximum((t - m) * lax.rsqrt(v + _EPS) * g_ref[...] + be_ref[...], 0.0)
    h2 = h2 + h1_ref[...]
    pooled = jnp.sum(h2, axis=0, keepdims=True)
    score_ref[...] = (
        jnp.dot(pooled, wp_ref[...], preferred_element_type=jnp.float32) + bp_ref[...]
    )


_prep_call = pl.pallas_call(
    _prep_body, out_shape=jax.ShapeDtypeStruct((_NP, _D), jnp.float32)
)
_dense0_call = pl.pallas_call(
    _dense0_body,
    out_shape=(
        jax.ShapeDtypeStruct((_N, _D), jnp.float32),
        jax.ShapeDtypeStruct((_NP, _D), jnp.float32),
    ),
)
_final_call = pl.pallas_call(
    _final_body, out_shape=jax.ShapeDtypeStruct((1, _D), jnp.float32)
)


def kernel(h, edge_index, W0, b0, g0, be0, W1, b1, g1, be1, Wp, bp):
    pad = jnp.full((_EP - _E,), _N, jnp.int32)
    src = jnp.concatenate([edge_index[0], pad])
    dst = jnp.concatenate([edge_index[1], pad])
    zeros_row = jnp.zeros((_RPT, _D), jnp.float32)

    src2 = src.reshape(_EP // _CH, _CH)
    dst2 = dst.reshape(_EP // _CH, _CH)

    degs = _deg_call(src, dst)
    x0 = _prep_call(h, degs)
    agg0p = _edge_kernel(x0, src2, dst2, zeros_row)
    h1, x1 = _dense0_call(agg0p, degs, W0, b0, g0, be0)
    agg1p = _edge_kernel(x1, src2, dst2, zeros_row)
    score = _final_call(agg1p, degs, W1, b1, g1, be1, h1, Wp, bp)
    return score


# speedup vs baseline: 3.2594x; 3.2594x over previous
import functools

import jax
import jax.numpy as jnp
from jax import lax
from jax.experimental import pallas as pl
from jax.experimental.pallas import tpu as pltpu
from jax.experimental.pallas import tpu_sc as plsc

_N = 10000
_E = 320000
_D = 128
_EPS = 1e-5

_NC = 2
_NS = 16
_NW = _NC * _NS
_NP = 10240
_RPT = _NP // _NS
_EP = 327680
_EPW = _EP // _NW
_CH = 128
_NCHUNK = _EPW // _CH

_MESH = plsc.VectorSubcoreMesh(
    core_axis_name="c", subcore_axis_name="s", num_cores=_NC, num_subcores=_NS
)


def _worker_ids():
    cid = lax.axis_index("c")
    sid = lax.axis_index("s")
    return cid, sid, sid * _NC + cid


@functools.partial(
    pl.kernel,
    out_type=jax.ShapeDtypeStruct((_NC, 2, _NP), jnp.float32),
    mesh=_MESH,
    scratch_types=[
        pltpu.VMEM((_CH,), jnp.int32),
        pltpu.VMEM((_CH,), jnp.int32),
        pltpu.VMEM((_CH,), jnp.float32),
        pltpu.VMEM_SHARED((_NP,), jnp.float32),
        pltpu.VMEM_SHARED((_NP,), jnp.float32),
    ],
)
def _deg_kernel(src_hbm, dst_hbm, ones_hbm, zeros_hbm, out_hbm,
                sidx, didx, ones_v, acc_s, acc_d):
    cid, sid, wid = _worker_ids()
    rows = pl.ds(sid * _RPT, _RPT)
    pltpu.sync_copy(zeros_hbm, acc_s.at[rows])
    pltpu.sync_copy(zeros_hbm, acc_d.at[rows])
    pltpu.sync_copy(ones_hbm, ones_v)
    plsc.subcore_barrier()

    def body(i, carry):
        off = pl.multiple_of(wid * _EPW + i * _CH, 128)
        pltpu.sync_copy(src_hbm.at[pl.ds(off, _CH)], sidx)
        pltpu.sync_copy(dst_hbm.at[pl.ds(off, _CH)], didx)
        pltpu.sync_copy(ones_v, acc_s.at[sidx], add=True)
        pltpu.sync_copy(ones_v, acc_d.at[didx], add=True)
        return carry

    lax.fori_loop(0, _NCHUNK, body, 0)
    plsc.subcore_barrier()
    pltpu.sync_copy(acc_s.at[rows], out_hbm.at[cid, 0, rows])
    pltpu.sync_copy(acc_d.at[rows], out_hbm.at[cid, 1, rows])


def _deg_call(src, dst):
    ones1 = jnp.ones((_CH,), jnp.float32)
    zeros1 = jnp.zeros((_RPT,), jnp.float32)
    degs = _deg_kernel(src, dst, ones1, zeros1)
    return degs[:, :, :, None]


_S0 = 80
_S1 = _NCHUNK * 2 - _S0
_G = 40


@functools.partial(
    pl.kernel,
    out_type=jax.ShapeDtypeStruct((_NC, _NP, _D), jnp.float32),
    mesh=_MESH,
    scratch_types=[
        pltpu.VMEM((_G, _CH), jnp.int32),
        pltpu.VMEM((_G, _CH), jnp.int32),
        pltpu.VMEM((_CH, _D), jnp.float32),
        pltpu.VMEM((_CH, _D), jnp.float32),
        pltpu.SemaphoreType.DMA,
        pltpu.SemaphoreType.DMA,
        pltpu.VMEM_SHARED((_NP, _D), jnp.float32),
    ],
)
def _edge_kernel(x_hbm, src_hbm, dst_hbm, zeros_hbm, out_hbm,
                 sidx, didx, rows0, rows1, sem0, sem1, acc):
    cid, sid, wid = _worker_ids()
    rows = pl.ds(sid * _RPT, _RPT)
    pltpu.sync_copy(zeros_hbm, acc.at[rows])
    plsc.subcore_barrier()

    def run_groups(tile_start, n_groups):
        for g in range(n_groups):
            chunks = pl.ds(tile_start + g * _G, _G)
            pltpu.sync_copy(src_hbm.at[chunks], sidx)
            pltpu.sync_copy(dst_hbm.at[chunks], didx)
            pltpu.async_copy(x_hbm.at[sidx.at[0]], rows0, sem0)

            def body(i, carry):
                i2 = i * 2
                pltpu.make_async_copy(
                    x_hbm.at[sidx.at[i2]], rows0, sem0).wait()
                pltpu.async_copy(x_hbm.at[sidx.at[i2 + 1]], rows1, sem1)
                pltpu.sync_copy(rows0, acc.at[didx.at[i2]], add=True)
                pltpu.make_async_copy(
                    x_hbm.at[sidx.at[i2 + 1]], rows1, sem1).wait()

                @pl.when(i2 + 2 < _G)
                def _():
                    pltpu.async_copy(x_hbm.at[sidx.at[i2 + 2]], rows0, sem0)

                pltpu.sync_copy(rows1, acc.at[didx.at[i2 + 1]], add=True)
                return carry

            lax.fori_loop(0, _G // 2, body, 0)

    @pl.when(cid == 0)
    def _():
        run_groups(sid * _S0, _S0 // _G)

    if _S1 > 0:
        @pl.when(cid == 1)
        def _():
            run_groups(16 * _S0 + sid * _S1, _S1 // _G)

    plsc.subcore_barrier()
    pltpu.sync_copy(acc.at[rows], out_hbm.at[cid, rows])


def _norms_from(degs, which):
    d = degs[0, which, : _N] + degs[1, which, : _N]
    return lax.rsqrt(jnp.maximum(d, 1.0))


def _prep_body(h_ref, degs_ref, x0_ref):
    degs = degs_ref[...]
    x0_ref[0:_N] = h_ref[...] * _norms_from(degs, 0)
    x0_ref[_N:] = jnp.zeros((_NP - _N, _D), jnp.float32)


def _dense0_body(aggp_ref, degs_ref, w_ref, b_ref, g_ref, be_ref,
                 h1_ref, x1_ref):
    degs = degs_ref[...]
    agg = (aggp_ref[0, : _N] + aggp_ref[1, : _N]) * _norms_from(degs, 1)
    t = jnp.dot(agg, w_ref[...], preferred_element_type=jnp.float32) + b_ref[...]
    m = jnp.mean(t, axis=0, keepdims=True)
    v = jnp.mean((t - m) ** 2, axis=0, keepdims=True)
    h1 = jnp.maximum((t - m) * lax.rsqrt(v + _EPS) * g_ref[...] + be_ref[...], 0.0)
    h1_ref[...] = h1
    x1_ref[0:_N] = h1 * _norms_from(degs, 0)
    x1_ref[_N:] = jnp.zeros((_NP - _N, _D), jnp.float32)


def _final_body(aggp_ref, degs_ref, w_ref, b_ref, g_ref, be_ref, h1_ref,
                wp_ref, bp_ref, score_ref):
    degs = degs_ref[...]
    agg = (aggp_ref[0, : _N] + aggp_ref[1, : _N]) * _norms_from(degs, 1)
    t = jnp.dot(agg, w_ref[...], preferred_element_type=jnp.float32) + b_ref[...]
    m = jnp.mean(t, axis=0, keepdims=True)
    v = jnp.mean((t - m) ** 2, axis=0, keepdims=True)
    h2 = jnp.maximum((t - m) * lax.rsqrt(v + _EPS) * g_ref[...] + be_ref[...], 0.0)
    h2 = h2 + h1_ref[...]
    pooled = jnp.sum(h2, axis=0, keepdims=True)
    score_ref[...] = (
        jnp.dot(pooled, wp_ref[...], preferred_element_type=jnp.float32) + bp_ref[...]
    )


_prep_call = pl.pallas_call(
    _prep_body, out_shape=jax.ShapeDtypeStruct((_NP, _D), jnp.float32)
)
_dense0_call = pl.pallas_call(
    _dense0_body,
    out_shape=(
        jax.ShapeDtypeStruct((_N, _D), jnp.float32),
        jax.ShapeDtypeStruct((_NP, _D), jnp.float32),
    ),
)
_final_call = pl.pallas_call(
    _final_body, out_shape=jax.ShapeDtypeStruct((1, _D), jnp.float32)
)


def kernel(h, edge_index, W0, b0, g0, be0, W1, b1, g1, be1, Wp, bp):
    pad = _N + (jnp.arange(_EP - _E, dtype=jnp.int32) % (_NP - _N))
    src = jnp.concatenate([edge_index[0], pad])
    dst = jnp.concatenate([edge_index[1], pad])
    zeros_row = jnp.zeros((_RPT, _D), jnp.float32)

    src2 = src.reshape(_EP // _CH, _CH)
    dst2 = dst.reshape(_EP // _CH, _CH)

    degs = _deg_call(src, dst)
    x0 = _prep_call(h, degs)
    agg0p = _edge_kernel(x0, src2, dst2, zeros_row)
    h1, x1 = _dense0_call(agg0p, degs, W0, b0, g0, be0)
    agg1p = _edge_kernel(x1, src2, dst2, zeros_row)
    score = _final_call(agg1p, degs, W1, b1, g1, be1, h1, Wp, bp)
    return score


# speedup vs baseline: 4.2627x; 1.3078x over previous
import functools

import jax
import jax.numpy as jnp
from jax import lax
from jax.experimental import pallas as pl
from jax.experimental.pallas import tpu as pltpu
from jax.experimental.pallas import tpu_sc as plsc

_N = 10000
_E = 320000
_D = 128
_EPS = 1e-5

_NC = 2
_NS = 16
_NW = _NC * _NS
_NP = 10240
_RPT = _NP // _NS
_EP = 327680
_EPW = _EP // _NW
_CH = 128
_NCHUNK = _EPW // _CH

_MESH = plsc.VectorSubcoreMesh(
    core_axis_name="c", subcore_axis_name="s", num_cores=_NC, num_subcores=_NS
)


def _worker_ids():
    cid = lax.axis_index("c")
    sid = lax.axis_index("s")
    return cid, sid, sid * _NC + cid


_K = 8


@functools.partial(
    pl.kernel,
    out_type=jax.ShapeDtypeStruct((_NC, 2, _NP), jnp.float32),
    mesh=_MESH,
    scratch_types=[
        pltpu.VMEM((_NCHUNK, _CH), jnp.int32),
        pltpu.VMEM((_NCHUNK, _CH), jnp.int32),
        pltpu.VMEM((_CH,), jnp.float32),
        pltpu.SemaphoreType.DMA,
        pltpu.SemaphoreType.DMA,
        pltpu.VMEM_SHARED((_NP,), jnp.float32),
        pltpu.VMEM_SHARED((_NP,), jnp.float32),
    ],
)
def _deg_kernel(src_hbm, dst_hbm, ones_hbm, zeros_hbm, out_hbm,
                sidx, didx, ones_v, ssem, dsem, acc_s, acc_d):
    cid, sid, wid = _worker_ids()
    rows = pl.ds(sid * _RPT, _RPT)
    chunks = pl.ds(wid * _NCHUNK, _NCHUNK)
    pltpu.sync_copy(zeros_hbm, acc_s.at[rows])
    pltpu.sync_copy(zeros_hbm, acc_d.at[rows])
    pltpu.sync_copy(ones_hbm, ones_v)
    pltpu.sync_copy(src_hbm.at[chunks], sidx)
    pltpu.sync_copy(dst_hbm.at[chunks], didx)
    plsc.subcore_barrier()

    def body(i, carry):
        pltpu.async_copy(ones_v, acc_s.at[sidx.at[i]], ssem, add=True)
        pltpu.async_copy(ones_v, acc_d.at[didx.at[i]], dsem, add=True)

        @pl.when(i >= _K)
        def _():
            pltpu.make_async_copy(ones_v, acc_s.at[sidx.at[0]], ssem).wait()
            pltpu.make_async_copy(ones_v, acc_d.at[didx.at[0]], dsem).wait()

        return carry

    lax.fori_loop(0, _NCHUNK, body, 0)
    for _ in range(_K):
        pltpu.make_async_copy(ones_v, acc_s.at[sidx.at[0]], ssem).wait()
        pltpu.make_async_copy(ones_v, acc_d.at[didx.at[0]], dsem).wait()
    plsc.subcore_barrier()
    pltpu.sync_copy(acc_s.at[rows], out_hbm.at[cid, 0, rows])
    pltpu.sync_copy(acc_d.at[rows], out_hbm.at[cid, 1, rows])


def _deg_call(src2, dst2):
    ones1 = jnp.ones((_CH,), jnp.float32)
    zeros1 = jnp.zeros((_RPT,), jnp.float32)
    return _deg_kernel(src2, dst2, ones1, zeros1)


_S0 = 80
_S1 = _NCHUNK * 2 - _S0
_G = 40


@functools.partial(
    pl.kernel,
    out_type=jax.ShapeDtypeStruct((_NC, _NP, _D), jnp.float32),
    mesh=_MESH,
    scratch_types=[
        pltpu.VMEM((_G, _CH), jnp.int32),
        pltpu.VMEM((_G, _CH), jnp.int32),
        pltpu.VMEM((_CH, _D), jnp.float32),
        pltpu.VMEM((_CH, _D), jnp.float32),
        pltpu.SemaphoreType.DMA,
        pltpu.SemaphoreType.DMA,
        pltpu.VMEM_SHARED((_NP, _D), jnp.float32),
    ],
)
def _edge_kernel(x_hbm, src_hbm, dst_hbm, zeros_hbm, out_hbm,
                 sidx, didx, rows0, rows1, sem0, sem1, acc):
    cid, sid, wid = _worker_ids()
    rows = pl.ds(sid * _RPT, _RPT)
    pltpu.sync_copy(zeros_hbm, acc.at[rows])
    plsc.subcore_barrier()

    def run_groups(tile_start, n_groups):
        for g in range(n_groups):
            chunks = pl.ds(tile_start + g * _G, _G)
            pltpu.sync_copy(src_hbm.at[chunks], sidx)
            pltpu.sync_copy(dst_hbm.at[chunks], didx)
            pltpu.async_copy(x_hbm.at[sidx.at[0]], rows0, sem0)

            def body(i, carry):
                i2 = i * 2
                pltpu.make_async_copy(
                    x_hbm.at[sidx.at[i2]], rows0, sem0).wait()
                pltpu.async_copy(x_hbm.at[sidx.at[i2 + 1]], rows1, sem1)
                pltpu.sync_copy(rows0, acc.at[didx.at[i2]], add=True)
                pltpu.make_async_copy(
                    x_hbm.at[sidx.at[i2 + 1]], rows1, sem1).wait()

                @pl.when(i2 + 2 < _G)
                def _():
                    pltpu.async_copy(x_hbm.at[sidx.at[i2 + 2]], rows0, sem0)

                pltpu.sync_copy(rows1, acc.at[didx.at[i2 + 1]], add=True)
                return carry

            lax.fori_loop(0, _G // 2, body, 0)

    @pl.when(cid == 0)
    def _():
        run_groups(sid * _S0, _S0 // _G)

    if _S1 > 0:
        @pl.when(cid == 1)
        def _():
            run_groups(16 * _S0 + sid * _S1, _S1 // _G)

    plsc.subcore_barrier()
    pltpu.sync_copy(acc.at[rows], out_hbm.at[cid, rows])


def _norms_from(degs, which):
    d = degs[0, which, : _N] + degs[1, which, : _N]
    return lax.rsqrt(jnp.maximum(d, 1.0))[:, None]


def _prep_body(h_ref, degs_ref, x0_ref):
    degs = degs_ref[...]
    x0_ref[0:_N] = h_ref[...] * _norms_from(degs, 0)
    x0_ref[_N:] = jnp.zeros((_NP - _N, _D), jnp.float32)


def _dense0_body(aggp_ref, degs_ref, w_ref, b_ref, g_ref, be_ref,
                 h1_ref, x1_ref):
    degs = degs_ref[...]
    agg = (aggp_ref[0, : _N] + aggp_ref[1, : _N]) * _norms_from(degs, 1)
    t = jnp.dot(agg, w_ref[...], preferred_element_type=jnp.float32) + b_ref[...]
    m = jnp.mean(t, axis=0, keepdims=True)
    v = jnp.mean((t - m) ** 2, axis=0, keepdims=True)
    h1 = jnp.maximum((t - m) * lax.rsqrt(v + _EPS) * g_ref[...] + be_ref[...], 0.0)
    h1_ref[...] = h1
    x1_ref[0:_N] = h1 * _norms_from(degs, 0)
    x1_ref[_N:] = jnp.zeros((_NP - _N, _D), jnp.float32)


def _final_body(aggp_ref, degs_ref, w_ref, b_ref, g_ref, be_ref, h1_ref,
                wp_ref, bp_ref, score_ref):
    degs = degs_ref[...]
    agg = (aggp_ref[0, : _N] + aggp_ref[1, : _N]) * _norms_from(degs, 1)
    t = jnp.dot(agg, w_ref[...], preferred_element_type=jnp.float32) + b_ref[...]
    m = jnp.mean(t, axis=0, keepdims=True)
    v = jnp.mean((t - m) ** 2, axis=0, keepdims=True)
    h2 = jnp.maximum((t - m) * lax.rsqrt(v + _EPS) * g_ref[...] + be_ref[...], 0.0)
    h2 = h2 + h1_ref[...]
    pooled = jnp.sum(h2, axis=0, keepdims=True)
    score_ref[...] = (
        jnp.dot(pooled, wp_ref[...], preferred_element_type=jnp.float32) + bp_ref[...]
    )


_prep_call = pl.pallas_call(
    _prep_body, out_shape=jax.ShapeDtypeStruct((_NP, _D), jnp.float32)
)
_dense0_call = pl.pallas_call(
    _dense0_body,
    out_shape=(
        jax.ShapeDtypeStruct((_N, _D), jnp.float32),
        jax.ShapeDtypeStruct((_NP, _D), jnp.float32),
    ),
)
_final_call = pl.pallas_call(
    _final_body, out_shape=jax.ShapeDtypeStruct((1, _D), jnp.float32)
)


def kernel(h, edge_index, W0, b0, g0, be0, W1, b1, g1, be1, Wp, bp):
    pad = _N + (jnp.arange(_EP - _E, dtype=jnp.int32) % (_NP - _N))
    src = jnp.concatenate([edge_index[0], pad])
    dst = jnp.concatenate([edge_index[1], pad])
    zeros_row = jnp.zeros((_RPT, _D), jnp.float32)

    src2 = src.reshape(_EP // _CH, _CH)
    dst2 = dst.reshape(_EP // _CH, _CH)

    degs = _deg_call(src2, dst2)
    x0 = _prep_call(h, degs)
    agg0p = _edge_kernel(x0, src2, dst2, zeros_row)
    h1, x1 = _dense0_call(agg0p, degs, W0, b0, g0, be0)
    agg1p = _edge_kernel(x1, src2, dst2, zeros_row)
    score = _final_call(agg1p, degs, W1, b1, g1, be1, h1, Wp, bp)
    return score
